# Initial kernel scaffold; baseline (speedup 1.0000x reference)
import jax
import jax.numpy as jnp
from jax.experimental import pallas as pl


def kernel(x, edge_index, edge_attr, Wf, bf, Ws, bs, Wffw, bffw, Wproj, bproj):
    raise NotImplementedError("write your pallas kernel here")


# speedup vs baseline: 1.8782x; 1.8782x over previous
import functools

import jax
import jax.numpy as jnp
from jax import lax
from jax.experimental import pallas as pl
from jax.experimental.pallas import tpu as pltpu
from jax.experimental.pallas import tpu_sc as plsc

N_NODES = 10000
N_EDGES = 160000
F_DIM = 256
DE_DIM = 16

NC = 2
NS = 16
NW = NC * NS


_ROWS_BLK = 1000


def _tables_body(x_ref, wd_ref, ws_ref, d_ref, s_ref):
    xb = x_ref[...]
    d_ref[...] = jnp.dot(xb, wd_ref[...], preferred_element_type=jnp.float32)
    s_ref[...] = jnp.dot(xb, ws_ref[...], preferred_element_type=jnp.float32)


def _node_tables(x, wd, ws):
    grid = (N_NODES // _ROWS_BLK,)
    return pl.pallas_call(
        _tables_body,
        grid=grid,
        in_specs=[
            pl.BlockSpec((_ROWS_BLK, F_DIM), lambda i: (i, 0)),
            pl.BlockSpec((F_DIM, 2 * F_DIM), lambda i: (0, 0)),
            pl.BlockSpec((F_DIM, 2 * F_DIM), lambda i: (0, 0)),
        ],
        out_specs=[
            pl.BlockSpec((_ROWS_BLK, 2 * F_DIM), lambda i: (i, 0)),
            pl.BlockSpec((_ROWS_BLK, 2 * F_DIM), lambda i: (i, 0)),
        ],
        out_shape=[
            jax.ShapeDtypeStruct((N_NODES, 2 * F_DIM), jnp.float32),
            jax.ShapeDtypeStruct((N_NODES, 2 * F_DIM), jnp.float32),
        ],
    )(x, wd, ws)


_EW = N_EDGES // NW
_GCH = 40
_GIT = _EW // _GCH


def _gather_kernel(d_hbm, s_hbm, dst_hbm, src_hbm, rd_hbm, rs_hbm,
                   dstv, srcv, dbuf, sbuf, semd, sems):
    wid = lax.axis_index("s") * NC + lax.axis_index("c")
    base = wid * _EW
    pltpu.sync_copy(dst_hbm.at[pl.ds(base, _EW)], dstv)
    pltpu.sync_copy(src_hbm.at[pl.ds(base, _EW)], srcv)

    def body(i, carry):
        off = i * _GCH
        cd = pltpu.async_copy(d_hbm.at[dstv.at[pl.ds(off, _GCH)]], dbuf, semd)
        cs = pltpu.async_copy(s_hbm.at[srcv.at[pl.ds(off, _GCH)]], sbuf, sems)
        cd.wait()
        pltpu.sync_copy(dbuf, rd_hbm.at[pl.ds(base + off, _GCH)])
        cs.wait()
        pltpu.sync_copy(sbuf, rs_hbm.at[pl.ds(base + off, _GCH)])
        return carry

    lax.fori_loop(0, _GIT, body, 0)


def _edge_gather(d, s, dst, src):
    k = functools.partial(
        pl.kernel,
        mesh=plsc.VectorSubcoreMesh(core_axis_name="c", subcore_axis_name="s"),
        out_type=[
            jax.ShapeDtypeStruct((N_EDGES, 2 * F_DIM), jnp.float32),
            jax.ShapeDtypeStruct((N_EDGES, 2 * F_DIM), jnp.float32),
        ],
        scratch_types=[
            pltpu.VMEM((_EW,), jnp.int32),
            pltpu.VMEM((_EW,), jnp.int32),
            pltpu.VMEM((_GCH, 2 * F_DIM), jnp.float32),
            pltpu.VMEM((_GCH, 2 * F_DIM), jnp.float32),
            pltpu.SemaphoreType.DMA,
            pltpu.SemaphoreType.DMA,
        ],
    )(_gather_kernel)
    return k(d, s, dst, src)


_EDGE_BLK = 1000


def _msg_body(rd_ref, rs_ref, ea_ref, we_ref, b_ref, m0_ref, m1_ref):
    pre = (rd_ref[...] + rs_ref[...]
           + jnp.dot(ea_ref[...], we_ref[...],
                     preferred_element_type=jnp.float32)
           + b_ref[...])
    g = pre[:, :F_DIM]
    c = pre[:, F_DIM:]
    gate = 1.0 / (1.0 + jnp.exp(-g))
    sp = jnp.maximum(c, 0.0) + jnp.log(1.0 + jnp.exp(-jnp.abs(c)))
    msg = gate * sp
    m0_ref[...] = msg[:, : F_DIM // 2]
    m1_ref[...] = msg[:, F_DIM // 2:]


def _edge_messages(rd, rs, ea, we, bcat):
    grid = (N_EDGES // _EDGE_BLK,)
    return pl.pallas_call(
        _msg_body,
        grid=grid,
        in_specs=[
            pl.BlockSpec((_EDGE_BLK, 2 * F_DIM), lambda i: (i, 0)),
            pl.BlockSpec((_EDGE_BLK, 2 * F_DIM), lambda i: (i, 0)),
            pl.BlockSpec((_EDGE_BLK, DE_DIM), lambda i: (i, 0)),
            pl.BlockSpec((DE_DIM, 2 * F_DIM), lambda i: (0, 0)),
            pl.BlockSpec((1, 2 * F_DIM), lambda i: (0, 0)),
        ],
        out_specs=[
            pl.BlockSpec((_EDGE_BLK, F_DIM // 2), lambda i: (i, 0)),
            pl.BlockSpec((_EDGE_BLK, F_DIM // 2), lambda i: (i, 0)),
        ],
        out_shape=[
            jax.ShapeDtypeStruct((N_EDGES, F_DIM // 2), jnp.float32),
            jax.ShapeDtypeStruct((N_EDGES, F_DIM // 2), jnp.float32),
        ],
    )(rd, rs, ea, we, bcat)


_SCH = 80
_ET = N_EDGES // NS
_SIT = _ET // _SCH
_ZROWS = 1000
_ZTILES = N_NODES // _ZROWS
_HF = F_DIM // 2


def _scatter_kernel(m0_hbm, m1_hbm, dst3d_hbm, z_hbm, a0_hbm, a1_hbm,
                    idxv, mbuf, acc_sh, sem):
    c = lax.axis_index("c")
    s = lax.axis_index("s")

    @pl.when(s < _ZTILES)
    def _():
        pltpu.sync_copy(z_hbm, acc_sh.at[pl.ds(s * _ZROWS, _ZROWS)])

    plsc.subcore_barrier()
    pltpu.sync_copy(dst3d_hbm.at[s], idxv)

    def run(m_hbm):
        def body(i, carry):
            pltpu.sync_copy(m_hbm.at[pl.ds(s * _ET + i * _SCH, _SCH)], mbuf)
            pltpu.sync_copy(mbuf, acc_sh.at[idxv.at[i]], add=True)
            return carry
        lax.fori_loop(0, _SIT, body, 0)

    @pl.when(c == 0)
    def _():
        run(m0_hbm)

    @pl.when(c == 1)
    def _():
        run(m1_hbm)

    plsc.subcore_barrier()

    @pl.when(jnp.logical_and(c == 0, s < _ZTILES))
    def _():
        pltpu.sync_copy(acc_sh.at[pl.ds(s * _ZROWS, _ZROWS)],
                        a0_hbm.at[pl.ds(s * _ZROWS, _ZROWS)])

    @pl.when(jnp.logical_and(c == 1, s < _ZTILES))
    def _():
        pltpu.sync_copy(acc_sh.at[pl.ds(s * _ZROWS, _ZROWS)],
                        a1_hbm.at[pl.ds(s * _ZROWS, _ZROWS)])


def _scatter_add(m0, m1, dst3d, zeros_blk):
    k = functools.partial(
        pl.kernel,
        mesh=plsc.VectorSubcoreMesh(core_axis_name="c", subcore_axis_name="s"),
        out_type=[
            jax.ShapeDtypeStruct((N_NODES, _HF), jnp.float32),
            jax.ShapeDtypeStruct((N_NODES, _HF), jnp.float32),
        ],
        scratch_types=[
            pltpu.VMEM((_SIT, _SCH), jnp.int32),
            pltpu.VMEM((_SCH, _HF), jnp.float32),
            pltpu.VMEM_SHARED((N_NODES, _HF), jnp.float32),
            pltpu.SemaphoreType.DMA,
        ],
    )(_scatter_kernel)
    return k(m0, m1, dst3d, zeros_blk)


def _mlp_body(x_ref, a0_ref, a1_ref, wffw_ref, bffw_ref, wproj_ref,
              bproj_ref, o_ref):
    h = x_ref[...] + jnp.concatenate([a0_ref[...], a1_ref[...]], axis=1)
    h = jnp.maximum(
        jnp.dot(h, wffw_ref[...], preferred_element_type=jnp.float32)
        + bffw_ref[...], 0.0)
    o_ref[...] = (jnp.dot(h, wproj_ref[...], preferred_element_type=jnp.float32)
                  + bproj_ref[...])


def _out_mlp(x, a0, a1, wffw, bffw, wproj, bproj):
    grid = (N_NODES // _ROWS_BLK,)
    return pl.pallas_call(
        _mlp_body,
        grid=grid,
        in_specs=[
            pl.BlockSpec((_ROWS_BLK, F_DIM), lambda i: (i, 0)),
            pl.BlockSpec((_ROWS_BLK, _HF), lambda i: (i, 0)),
            pl.BlockSpec((_ROWS_BLK, _HF), lambda i: (i, 0)),
            pl.BlockSpec((F_DIM, F_DIM), lambda i: (0, 0)),
            pl.BlockSpec((1, F_DIM), lambda i: (0, 0)),
            pl.BlockSpec((F_DIM, F_DIM), lambda i: (0, 0)),
            pl.BlockSpec((1, F_DIM), lambda i: (0, 0)),
        ],
        out_specs=pl.BlockSpec((_ROWS_BLK, F_DIM), lambda i: (i, 0)),
        out_shape=jax.ShapeDtypeStruct((N_NODES, F_DIM), jnp.float32),
    )(x, a0, a1, wffw, bffw, wproj, bproj)


def kernel(x, edge_index, edge_attr, Wf, bf, Ws, bs, Wffw, bffw, Wproj, bproj):
    src = edge_index[0]
    dst = edge_index[1]
    wd = jnp.concatenate([Wf[:F_DIM], Ws[:F_DIM]], axis=1)
    ws_ = jnp.concatenate([Wf[F_DIM:2 * F_DIM], Ws[F_DIM:2 * F_DIM]], axis=1)
    we = jnp.concatenate([Wf[2 * F_DIM:], Ws[2 * F_DIM:]], axis=1)
    bcat = jnp.concatenate([bf, bs]).reshape(1, 2 * F_DIM)

    d_tab, s_tab = _node_tables(x, wd, ws_)
    rd, rs = _edge_gather(d_tab, s_tab, dst, src)
    m0, m1 = _edge_messages(rd, rs, edge_attr, we, bcat)
    dst3d = dst.reshape(NS, _SIT, _SCH)
    zeros_blk = jnp.zeros((_ZROWS, _HF), jnp.float32)
    a0, a1 = _scatter_add(m0, m1, dst3d, zeros_blk)
    return _out_mlp(x, a0, a1, bffw=bffw.reshape(1, F_DIM), wffw=Wffw,
                    wproj=Wproj, bproj=bproj.reshape(1, F_DIM))


# speedup vs baseline: 2.1173x; 1.1273x over previous
import functools

import jax
import jax.numpy as jnp
from jax import lax
from jax.experimental import pallas as pl
from jax.experimental.pallas import tpu as pltpu
from jax.experimental.pallas import tpu_sc as plsc

N_NODES = 10000
N_EDGES = 160000
F_DIM = 256
DE_DIM = 16

NC = 2
NS = 16
NW = NC * NS


_ROWS_BLK = 1000


def _tables_body(x_ref, wd_ref, ws_ref, d_ref, s_ref):
    xb = x_ref[...]
    d_ref[...] = jnp.dot(xb, wd_ref[...], preferred_element_type=jnp.float32)
    s_ref[...] = jnp.dot(xb, ws_ref[...], preferred_element_type=jnp.float32)


def _node_tables(x, wd, ws):
    grid = (N_NODES // _ROWS_BLK,)
    return pl.pallas_call(
        _tables_body,
        grid=grid,
        in_specs=[
            pl.BlockSpec((_ROWS_BLK, F_DIM), lambda i: (i, 0)),
            pl.BlockSpec((F_DIM, 2 * F_DIM), lambda i: (0, 0)),
            pl.BlockSpec((F_DIM, 2 * F_DIM), lambda i: (0, 0)),
        ],
        out_specs=[
            pl.BlockSpec((_ROWS_BLK, 2 * F_DIM), lambda i: (i, 0)),
            pl.BlockSpec((_ROWS_BLK, 2 * F_DIM), lambda i: (i, 0)),
        ],
        out_shape=[
            jax.ShapeDtypeStruct((N_NODES, 2 * F_DIM), jnp.float32),
            jax.ShapeDtypeStruct((N_NODES, 2 * F_DIM), jnp.float32),
        ],
    )(x, wd, ws)


_EW = N_EDGES // NW
_GCH = 40
_GIT = _EW // _GCH


def _gather_kernel(d_hbm, s_hbm, dst_hbm, src_hbm, rd_hbm, rs_hbm,
                   dstv, srcv, dbuf0, dbuf1, sbuf0, sbuf1,
                   semd0, semd1, sems0, sems1):
    wid = lax.axis_index("s") * NC + lax.axis_index("c")
    base = wid * _EW
    pltpu.sync_copy(dst_hbm.at[pl.ds(base, _EW)], dstv)
    pltpu.sync_copy(src_hbm.at[pl.ds(base, _EW)], srcv)

    dbufs = (dbuf0, dbuf1)
    sbufs = (sbuf0, sbuf1)
    semds = (semd0, semd1)
    semss = (sems0, sems1)

    def start(b, i):
        off = i * _GCH
        pltpu.async_copy(d_hbm.at[dstv.at[pl.ds(off, _GCH)]],
                         dbufs[b], semds[b])
        pltpu.async_copy(s_hbm.at[srcv.at[pl.ds(off, _GCH)]],
                         sbufs[b], semss[b])

    def finish(b, i):
        off = i * _GCH
        pltpu.make_async_copy(d_hbm.at[dstv.at[pl.ds(off, _GCH)]],
                              dbufs[b], semds[b]).wait()
        pltpu.sync_copy(dbufs[b], rd_hbm.at[pl.ds(base + off, _GCH)])
        pltpu.make_async_copy(s_hbm.at[srcv.at[pl.ds(off, _GCH)]],
                              sbufs[b], semss[b]).wait()
        pltpu.sync_copy(sbufs[b], rs_hbm.at[pl.ds(base + off, _GCH)])

    start(0, 0)
    start(1, 1)

    def body(g, carry):
        for b in range(2):
            i = 2 * g + b
            finish(b, i)

            @pl.when(i + 2 < _GIT)
            def _():
                start(b, i + 2)
        return carry

    lax.fori_loop(0, _GIT // 2, body, 0)
    if _GIT % 2:
        finish(0, _GIT - 1)


def _edge_gather(d, s, dst, src):
    k = functools.partial(
        pl.kernel,
        mesh=plsc.VectorSubcoreMesh(core_axis_name="c", subcore_axis_name="s"),
        out_type=[
            jax.ShapeDtypeStruct((N_EDGES, 2 * F_DIM), jnp.float32),
            jax.ShapeDtypeStruct((N_EDGES, 2 * F_DIM), jnp.float32),
        ],
        scratch_types=[
            pltpu.VMEM((_EW,), jnp.int32),
            pltpu.VMEM((_EW,), jnp.int32),
            pltpu.VMEM((_GCH, 2 * F_DIM), jnp.float32),
            pltpu.VMEM((_GCH, 2 * F_DIM), jnp.float32),
            pltpu.VMEM((_GCH, 2 * F_DIM), jnp.float32),
            pltpu.VMEM((_GCH, 2 * F_DIM), jnp.float32),
            pltpu.SemaphoreType.DMA,
            pltpu.SemaphoreType.DMA,
            pltpu.SemaphoreType.DMA,
            pltpu.SemaphoreType.DMA,
        ],
    )(_gather_kernel)
    return k(d, s, dst, src)


_EDGE_BLK = 1000


def _msg_body(rd_ref, rs_ref, ea_ref, we_ref, b_ref, m0_ref, m1_ref):
    pre = (rd_ref[...] + rs_ref[...]
           + jnp.dot(ea_ref[...], we_ref[...],
                     preferred_element_type=jnp.float32)
           + b_ref[...])
    g = pre[:, :F_DIM]
    c = pre[:, F_DIM:]
    gate = 1.0 / (1.0 + jnp.exp(-g))
    sp = jnp.maximum(c, 0.0) + jnp.log(1.0 + jnp.exp(-jnp.abs(c)))
    msg = gate * sp
    m0_ref[...] = msg[:, : F_DIM // 2]
    m1_ref[...] = msg[:, F_DIM // 2:]


def _edge_messages(rd, rs, ea, we, bcat):
    grid = (N_EDGES // _EDGE_BLK,)
    return pl.pallas_call(
        _msg_body,
        grid=grid,
        in_specs=[
            pl.BlockSpec((_EDGE_BLK, 2 * F_DIM), lambda i: (i, 0)),
            pl.BlockSpec((_EDGE_BLK, 2 * F_DIM), lambda i: (i, 0)),
            pl.BlockSpec((_EDGE_BLK, DE_DIM), lambda i: (i, 0)),
            pl.BlockSpec((DE_DIM, 2 * F_DIM), lambda i: (0, 0)),
            pl.BlockSpec((1, 2 * F_DIM), lambda i: (0, 0)),
        ],
        out_specs=[
            pl.BlockSpec((_EDGE_BLK, F_DIM // 2), lambda i: (i, 0)),
            pl.BlockSpec((_EDGE_BLK, F_DIM // 2), lambda i: (i, 0)),
        ],
        out_shape=[
            jax.ShapeDtypeStruct((N_EDGES, F_DIM // 2), jnp.float32),
            jax.ShapeDtypeStruct((N_EDGES, F_DIM // 2), jnp.float32),
        ],
    )(rd, rs, ea, we, bcat)


_SCH = 80
_ET = N_EDGES // NS
_SIT = _ET // _SCH
_ZROWS = 1000
_ZTILES = N_NODES // _ZROWS
_HF = F_DIM // 2


def _scatter_kernel(m0_hbm, m1_hbm, dst3d_hbm, z_hbm, a0_hbm, a1_hbm,
                    idxv, mbuf0, mbuf1, acc_sh, seml0, seml1, semc0, semc1):
    c = lax.axis_index("c")
    s = lax.axis_index("s")

    @pl.when(s < _ZTILES)
    def _():
        pltpu.sync_copy(z_hbm, acc_sh.at[pl.ds(s * _ZROWS, _ZROWS)])

    plsc.subcore_barrier()
    pltpu.sync_copy(dst3d_hbm.at[s], idxv)

    mbufs = (mbuf0, mbuf1)
    semls = (seml0, seml1)
    semcs = (semc0, semc1)

    def run(m_hbm):
        def startload(b, i):
            pltpu.async_copy(m_hbm.at[pl.ds(s * _ET + i * _SCH, _SCH)],
                             mbufs[b], semls[b])

        def finish(b, i):
            pltpu.make_async_copy(
                m_hbm.at[pl.ds(s * _ET + i * _SCH, _SCH)],
                mbufs[b], semls[b]).wait()
            pltpu.async_copy(mbufs[b], acc_sh.at[idxv.at[i]], semcs[b],
                             add=True)
            pltpu.make_async_copy(mbufs[b], acc_sh.at[idxv.at[i]],
                                  semcs[b]).wait()

        startload(0, 0)
        startload(1, 1)

        def body(g, carry):
            for b in range(2):
                i = 2 * g + b
                finish(b, i)

                @pl.when(i + 2 < _SIT)
                def _():
                    startload(b, i + 2)
            return carry

        lax.fori_loop(0, _SIT // 2, body, 0)
        if _SIT % 2:
            finish(0, _SIT - 1)

    @pl.when(c == 0)
    def _():
        run(m0_hbm)

    @pl.when(c == 1)
    def _():
        run(m1_hbm)

    plsc.subcore_barrier()

    @pl.when(jnp.logical_and(c == 0, s < _ZTILES))
    def _():
        pltpu.sync_copy(acc_sh.at[pl.ds(s * _ZROWS, _ZROWS)],
                        a0_hbm.at[pl.ds(s * _ZROWS, _ZROWS)])

    @pl.when(jnp.logical_and(c == 1, s < _ZTILES))
    def _():
        pltpu.sync_copy(acc_sh.at[pl.ds(s * _ZROWS, _ZROWS)],
                        a1_hbm.at[pl.ds(s * _ZROWS, _ZROWS)])


def _scatter_add(m0, m1, dst3d, zeros_blk):
    k = functools.partial(
        pl.kernel,
        mesh=plsc.VectorSubcoreMesh(core_axis_name="c", subcore_axis_name="s"),
        out_type=[
            jax.ShapeDtypeStruct((N_NODES, _HF), jnp.float32),
            jax.ShapeDtypeStruct((N_NODES, _HF), jnp.float32),
        ],
        scratch_types=[
            pltpu.VMEM((_SIT, _SCH), jnp.int32),
            pltpu.VMEM((_SCH, _HF), jnp.float32),
            pltpu.VMEM((_SCH, _HF), jnp.float32),
            pltpu.VMEM_SHARED((N_NODES, _HF), jnp.float32),
            pltpu.SemaphoreType.DMA,
            pltpu.SemaphoreType.DMA,
            pltpu.SemaphoreType.DMA,
            pltpu.SemaphoreType.DMA,
        ],
    )(_scatter_kernel)
    return k(m0, m1, dst3d, zeros_blk)


def _mlp_body(x_ref, a0_ref, a1_ref, wffw_ref, bffw_ref, wproj_ref,
              bproj_ref, o_ref):
    h = x_ref[...] + jnp.concatenate([a0_ref[...], a1_ref[...]], axis=1)
    h = jnp.maximum(
        jnp.dot(h, wffw_ref[...], preferred_element_type=jnp.float32)
        + bffw_ref[...], 0.0)
    o_ref[...] = (jnp.dot(h, wproj_ref[...], preferred_element_type=jnp.float32)
                  + bproj_ref[...])


def _out_mlp(x, a0, a1, wffw, bffw, wproj, bproj):
    grid = (N_NODES // _ROWS_BLK,)
    return pl.pallas_call(
        _mlp_body,
        grid=grid,
        in_specs=[
            pl.BlockSpec((_ROWS_BLK, F_DIM), lambda i: (i, 0)),
            pl.BlockSpec((_ROWS_BLK, _HF), lambda i: (i, 0)),
            pl.BlockSpec((_ROWS_BLK, _HF), lambda i: (i, 0)),
            pl.BlockSpec((F_DIM, F_DIM), lambda i: (0, 0)),
            pl.BlockSpec((1, F_DIM), lambda i: (0, 0)),
            pl.BlockSpec((F_DIM, F_DIM), lambda i: (0, 0)),
            pl.BlockSpec((1, F_DIM), lambda i: (0, 0)),
        ],
        out_specs=pl.BlockSpec((_ROWS_BLK, F_DIM), lambda i: (i, 0)),
        out_shape=jax.ShapeDtypeStruct((N_NODES, F_DIM), jnp.float32),
    )(x, a0, a1, wffw, bffw, wproj, bproj)


def kernel(x, edge_index, edge_attr, Wf, bf, Ws, bs, Wffw, bffw, Wproj, bproj):
    src = edge_index[0]
    dst = edge_index[1]
    wd = jnp.concatenate([Wf[:F_DIM], Ws[:F_DIM]], axis=1)
    ws_ = jnp.concatenate([Wf[F_DIM:2 * F_DIM], Ws[F_DIM:2 * F_DIM]], axis=1)
    we = jnp.concatenate([Wf[2 * F_DIM:], Ws[2 * F_DIM:]], axis=1)
    bcat = jnp.concatenate([bf, bs]).reshape(1, 2 * F_DIM)

    d_tab, s_tab = _node_tables(x, wd, ws_)
    rd, rs = _edge_gather(d_tab, s_tab, dst, src)
    m0, m1 = _edge_messages(rd, rs, edge_attr, we, bcat)
    dst3d = dst.reshape(NS, _SIT, _SCH)
    zeros_blk = jnp.zeros((_ZROWS, _HF), jnp.float32)
    a0, a1 = _scatter_add(m0, m1, dst3d, zeros_blk)
    return _out_mlp(x, a0, a1, bffw=bffw.reshape(1, F_DIM), wffw=Wffw,
                    wproj=Wproj, bproj=bproj.reshape(1, F_DIM))


# speedup vs baseline: 3.1521x; 1.4888x over previous
import functools

import jax
import jax.numpy as jnp
from jax import lax
from jax.experimental import pallas as pl
from jax.experimental.pallas import tpu as pltpu
from jax.experimental.pallas import tpu_sc as plsc

N_NODES = 10000
N_EDGES = 160000
F_DIM = 256
DE_DIM = 16

NC = 2
NS = 16
NW = NC * NS


_ROWS_BLK = 1000


def _rne16(x):
    b = jax.lax.bitcast_convert_type(x, jnp.int32)
    r = b + 0x7FFF + jax.lax.shift_right_logical(b, 16) % 2
    return jax.lax.shift_right_logical(r, 16)


def _pack2(gate_f32, core_f32):
    return _rne16(gate_f32) | (_rne16(core_f32) << 16)


def _unpack_lo(w):
    return jax.lax.bitcast_convert_type(w << 16, jnp.float32)


def _unpack_hi(w):
    return jax.lax.bitcast_convert_type(w & jnp.int32(-65536), jnp.float32)


def _tables_body(x_ref, wfd_ref, wsd_ref, wfs_ref, wss_ref, d_ref, s_ref):
    xb = x_ref[...]
    d_ref[...] = _pack2(
        jnp.dot(xb, wfd_ref[...], preferred_element_type=jnp.float32),
        jnp.dot(xb, wsd_ref[...], preferred_element_type=jnp.float32))
    s_ref[...] = _pack2(
        jnp.dot(xb, wfs_ref[...], preferred_element_type=jnp.float32),
        jnp.dot(xb, wss_ref[...], preferred_element_type=jnp.float32))


def _node_tables(x, wfd, wsd, wfs, wss):
    grid = (N_NODES // _ROWS_BLK,)
    wspec = pl.BlockSpec((F_DIM, F_DIM), lambda i: (0, 0))
    return pl.pallas_call(
        _tables_body,
        grid=grid,
        in_specs=[
            pl.BlockSpec((_ROWS_BLK, F_DIM), lambda i: (i, 0)),
            wspec, wspec, wspec, wspec,
        ],
        out_specs=[
            pl.BlockSpec((_ROWS_BLK, F_DIM), lambda i: (i, 0)),
            pl.BlockSpec((_ROWS_BLK, F_DIM), lambda i: (i, 0)),
        ],
        out_shape=[
            jax.ShapeDtypeStruct((N_NODES, F_DIM), jnp.int32),
            jax.ShapeDtypeStruct((N_NODES, F_DIM), jnp.int32),
        ],
    )(x, wfd, wsd, wfs, wss)


_EW = N_EDGES // NS
_GCH = 80
_GIT = _EW // _GCH


def _gather_kernel(d_hbm, s_hbm, dst_hbm, src_hbm, rd_hbm, rs_hbm,
                   idxv, buf0, buf1, sem0, sem1):
    c = lax.axis_index("c")
    s = lax.axis_index("s")
    base = s * _EW
    bufs = (buf0, buf1)
    sems = (sem0, sem1)

    def run(tab_hbm, idx_hbm, out_hbm):
        pltpu.sync_copy(idx_hbm.at[pl.ds(base, _EW)], idxv)

        def start(b, i):
            pltpu.async_copy(tab_hbm.at[idxv.at[pl.ds(i * _GCH, _GCH)]],
                             bufs[b], sems[b])

        def finish(b, i):
            pltpu.make_async_copy(
                tab_hbm.at[idxv.at[pl.ds(i * _GCH, _GCH)]],
                bufs[b], sems[b]).wait()
            pltpu.sync_copy(bufs[b], out_hbm.at[pl.ds(base + i * _GCH, _GCH)])

        start(0, 0)
        start(1, 1)

        def body(g, carry):
            for b in range(2):
                i = 2 * g + b
                finish(b, i)

                @pl.when(i + 2 < _GIT)
                def _():
                    start(b, i + 2)
            return carry

        lax.fori_loop(0, _GIT // 2, body, 0)
        if _GIT % 2:
            finish(0, _GIT - 1)

    @pl.when(c == 0)
    def _():
        run(d_hbm, dst_hbm, rd_hbm)

    @pl.when(c == 1)
    def _():
        run(s_hbm, src_hbm, rs_hbm)


def _edge_gather(d, s, dst, src):
    k = functools.partial(
        pl.kernel,
        mesh=plsc.VectorSubcoreMesh(core_axis_name="c", subcore_axis_name="s"),
        out_type=[
            jax.ShapeDtypeStruct((N_EDGES, F_DIM), jnp.int32),
            jax.ShapeDtypeStruct((N_EDGES, F_DIM), jnp.int32),
        ],
        scratch_types=[
            pltpu.VMEM((_EW,), jnp.int32),
            pltpu.VMEM((_GCH, F_DIM), jnp.int32),
            pltpu.VMEM((_GCH, F_DIM), jnp.int32),
            pltpu.SemaphoreType.DMA,
            pltpu.SemaphoreType.DMA,
        ],
    )(_gather_kernel)
    return k(d, s, dst, src)


_EDGE_BLK = 2000


def _msg_body(rd_ref, rs_ref, ea_ref, weg_ref, wec_ref, bg_ref, bc_ref,
              m0_ref, m1_ref):
    wd = rd_ref[...]
    ws = rs_ref[...]
    ea = ea_ref[...]
    g = (_unpack_lo(wd) + _unpack_lo(ws)
         + jnp.dot(ea, weg_ref[...], preferred_element_type=jnp.float32)
         + bg_ref[...])
    c = (_unpack_hi(wd) + _unpack_hi(ws)
         + jnp.dot(ea, wec_ref[...], preferred_element_type=jnp.float32)
         + bc_ref[...])
    gate = 1.0 / (1.0 + jnp.exp(-g))
    sp = jnp.maximum(c, 0.0) + jnp.log(1.0 + jnp.exp(-jnp.abs(c)))
    msg = gate * sp
    m0_ref[...] = msg[:, : F_DIM // 2]
    m1_ref[...] = msg[:, F_DIM // 2:]


def _edge_messages(rd, rs, ea, weg, wec, bg, bc):
    grid = (N_EDGES // _EDGE_BLK,)
    wspec = pl.BlockSpec((DE_DIM, F_DIM), lambda i: (0, 0))
    bspec = pl.BlockSpec((1, F_DIM), lambda i: (0, 0))
    return pl.pallas_call(
        _msg_body,
        grid=grid,
        in_specs=[
            pl.BlockSpec((_EDGE_BLK, F_DIM), lambda i: (i, 0)),
            pl.BlockSpec((_EDGE_BLK, F_DIM), lambda i: (i, 0)),
            pl.BlockSpec((_EDGE_BLK, DE_DIM), lambda i: (i, 0)),
            wspec, wspec, bspec, bspec,
        ],
        out_specs=[
            pl.BlockSpec((_EDGE_BLK, F_DIM // 2), lambda i: (i, 0)),
            pl.BlockSpec((_EDGE_BLK, F_DIM // 2), lambda i: (i, 0)),
        ],
        out_shape=[
            jax.ShapeDtypeStruct((N_EDGES, F_DIM // 2), jnp.float32),
            jax.ShapeDtypeStruct((N_EDGES, F_DIM // 2), jnp.float32),
        ],
    )(rd, rs, ea, weg, wec, bg, bc)


_SCH = 80
_ET = N_EDGES // NS
_SIT = _ET // _SCH
_ZROWS = 1000
_ZTILES = N_NODES // _ZROWS
_HF = F_DIM // 2


def _scatter_kernel(m0_hbm, m1_hbm, dst3d_hbm, z_hbm, a0_hbm, a1_hbm,
                    idxv, mbuf0, mbuf1, acc_sh, seml0, seml1, semc0, semc1):
    c = lax.axis_index("c")
    s = lax.axis_index("s")

    @pl.when(s < _ZTILES)
    def _():
        pltpu.sync_copy(z_hbm, acc_sh.at[pl.ds(s * _ZROWS, _ZROWS)])

    plsc.subcore_barrier()
    pltpu.sync_copy(dst3d_hbm.at[s], idxv)

    mbufs = (mbuf0, mbuf1)
    semls = (seml0, seml1)
    semcs = (semc0, semc1)

    def run(m_hbm):
        def startload(b, i):
            pltpu.async_copy(m_hbm.at[pl.ds(s * _ET + i * _SCH, _SCH)],
                             mbufs[b], semls[b])

        def finish(b, i):
            pltpu.make_async_copy(
                m_hbm.at[pl.ds(s * _ET + i * _SCH, _SCH)],
                mbufs[b], semls[b]).wait()
            pltpu.async_copy(mbufs[b], acc_sh.at[idxv.at[i]], semcs[b],
                             add=True)
            pltpu.make_async_copy(mbufs[b], acc_sh.at[idxv.at[i]],
                                  semcs[b]).wait()

        startload(0, 0)
        startload(1, 1)

        def body(g, carry):
            for b in range(2):
                i = 2 * g + b
                finish(b, i)

                @pl.when(i + 2 < _SIT)
                def _():
                    startload(b, i + 2)
            return carry

        lax.fori_loop(0, _SIT // 2, body, 0)
        if _SIT % 2:
            finish(0, _SIT - 1)

    @pl.when(c == 0)
    def _():
        run(m0_hbm)

    @pl.when(c == 1)
    def _():
        run(m1_hbm)

    plsc.subcore_barrier()

    @pl.when(jnp.logical_and(c == 0, s < _ZTILES))
    def _():
        pltpu.sync_copy(acc_sh.at[pl.ds(s * _ZROWS, _ZROWS)],
                        a0_hbm.at[pl.ds(s * _ZROWS, _ZROWS)])

    @pl.when(jnp.logical_and(c == 1, s < _ZTILES))
    def _():
        pltpu.sync_copy(acc_sh.at[pl.ds(s * _ZROWS, _ZROWS)],
                        a1_hbm.at[pl.ds(s * _ZROWS, _ZROWS)])


def _scatter_add(m0, m1, dst3d, zeros_blk):
    k = functools.partial(
        pl.kernel,
        mesh=plsc.VectorSubcoreMesh(core_axis_name="c", subcore_axis_name="s"),
        out_type=[
            jax.ShapeDtypeStruct((N_NODES, _HF), jnp.float32),
            jax.ShapeDtypeStruct((N_NODES, _HF), jnp.float32),
        ],
        scratch_types=[
            pltpu.VMEM((_SIT, _SCH), jnp.int32),
            pltpu.VMEM((_SCH, _HF), jnp.float32),
            pltpu.VMEM((_SCH, _HF), jnp.float32),
            pltpu.VMEM_SHARED((N_NODES, _HF), jnp.float32),
            pltpu.SemaphoreType.DMA,
            pltpu.SemaphoreType.DMA,
            pltpu.SemaphoreType.DMA,
            pltpu.SemaphoreType.DMA,
        ],
    )(_scatter_kernel)
    return k(m0, m1, dst3d, zeros_blk)


def _mlp_body(x_ref, a0_ref, a1_ref, wffw_ref, bffw_ref, wproj_ref,
              bproj_ref, o_ref):
    h = x_ref[...] + jnp.concatenate([a0_ref[...], a1_ref[...]], axis=1)
    h = jnp.maximum(
        jnp.dot(h, wffw_ref[...], preferred_element_type=jnp.float32)
        + bffw_ref[...], 0.0)
    o_ref[...] = (jnp.dot(h, wproj_ref[...], preferred_element_type=jnp.float32)
                  + bproj_ref[...])


def _out_mlp(x, a0, a1, wffw, bffw, wproj, bproj):
    grid = (N_NODES // _ROWS_BLK,)
    return pl.pallas_call(
        _mlp_body,
        grid=grid,
        in_specs=[
            pl.BlockSpec((_ROWS_BLK, F_DIM), lambda i: (i, 0)),
            pl.BlockSpec((_ROWS_BLK, _HF), lambda i: (i, 0)),
            pl.BlockSpec((_ROWS_BLK, _HF), lambda i: (i, 0)),
            pl.BlockSpec((F_DIM, F_DIM), lambda i: (0, 0)),
            pl.BlockSpec((1, F_DIM), lambda i: (0, 0)),
            pl.BlockSpec((F_DIM, F_DIM), lambda i: (0, 0)),
            pl.BlockSpec((1, F_DIM), lambda i: (0, 0)),
        ],
        out_specs=pl.BlockSpec((_ROWS_BLK, F_DIM), lambda i: (i, 0)),
        out_shape=jax.ShapeDtypeStruct((N_NODES, F_DIM), jnp.float32),
    )(x, a0, a1, wffw, bffw, wproj, bproj)


def kernel(x, edge_index, edge_attr, Wf, bf, Ws, bs, Wffw, bffw, Wproj, bproj):
    src = edge_index[0]
    dst = edge_index[1]
    d_tab, s_tab = _node_tables(x, Wf[:F_DIM], Ws[:F_DIM],
                                Wf[F_DIM:2 * F_DIM], Ws[F_DIM:2 * F_DIM])
    rd, rs = _edge_gather(d_tab, s_tab, dst, src)
    m0, m1 = _edge_messages(rd, rs, edge_attr,
                            Wf[2 * F_DIM:], Ws[2 * F_DIM:],
                            bf.reshape(1, F_DIM), bs.reshape(1, F_DIM))
    dst3d = dst.reshape(NS, _SIT, _SCH)
    zeros_blk = jnp.zeros((_ZROWS, _HF), jnp.float32)
    a0, a1 = _scatter_add(m0, m1, dst3d, zeros_blk)
    return _out_mlp(x, a0, a1, bffw=bffw.reshape(1, F_DIM), wffw=Wffw,
                    wproj=Wproj, bproj=bproj.reshape(1, F_DIM))


# speedup vs baseline: 3.1617x; 1.0031x over previous
import functools

import jax
import jax.numpy as jnp
from jax import lax
from jax.experimental import pallas as pl
from jax.experimental.pallas import tpu as pltpu
from jax.experimental.pallas import tpu_sc as plsc

N_NODES = 10000
N_EDGES = 160000
F_DIM = 256
DE_DIM = 16

NC = 2
NS = 16
NW = NC * NS


_ROWS_BLK = 1000


def _rne16(x):
    b = jax.lax.bitcast_convert_type(x, jnp.int32)
    r = b + 0x7FFF + jax.lax.shift_right_logical(b, 16) % 2
    return jax.lax.shift_right_logical(r, 16)


def _pack2(gate_f32, core_f32):
    return _rne16(gate_f32) | (_rne16(core_f32) << 16)


def _unpack_lo(w):
    return jax.lax.bitcast_convert_type(w << 16, jnp.float32)


def _unpack_hi(w):
    return jax.lax.bitcast_convert_type(w & jnp.int32(-65536), jnp.float32)


def _tables_body(x_ref, wfd_ref, wsd_ref, wfs_ref, wss_ref, d_ref, s_ref):
    xb = x_ref[...]
    d_ref[...] = _pack2(
        jnp.dot(xb, wfd_ref[...], preferred_element_type=jnp.float32),
        jnp.dot(xb, wsd_ref[...], preferred_element_type=jnp.float32))
    s_ref[...] = _pack2(
        jnp.dot(xb, wfs_ref[...], preferred_element_type=jnp.float32),
        jnp.dot(xb, wss_ref[...], preferred_element_type=jnp.float32))


def _node_tables(x, wfd, wsd, wfs, wss):
    grid = (N_NODES // _ROWS_BLK,)
    wspec = pl.BlockSpec((F_DIM, F_DIM), lambda i: (0, 0))
    return pl.pallas_call(
        _tables_body,
        grid=grid,
        in_specs=[
            pl.BlockSpec((_ROWS_BLK, F_DIM), lambda i: (i, 0)),
            wspec, wspec, wspec, wspec,
        ],
        out_specs=[
            pl.BlockSpec((_ROWS_BLK, F_DIM), lambda i: (i, 0)),
            pl.BlockSpec((_ROWS_BLK, F_DIM), lambda i: (i, 0)),
        ],
        out_shape=[
            jax.ShapeDtypeStruct((N_NODES, F_DIM), jnp.int32),
            jax.ShapeDtypeStruct((N_NODES, F_DIM), jnp.int32),
        ],
    )(x, wfd, wsd, wfs, wss)


def _make_edge_gather(n_edges, gch):
    ew = n_edges // NS
    git = ew // gch

    def _gather_kernel(d_hbm, s_hbm, dst_hbm, src_hbm, rd_hbm, rs_hbm,
                       idxv, buf0, buf1, sem0, sem1):
        c = lax.axis_index("c")
        s = lax.axis_index("s")
        base = s * ew
        bufs = (buf0, buf1)
        sems = (sem0, sem1)

        def run(tab_hbm, idx_hbm, out_hbm):
            pltpu.sync_copy(idx_hbm.at[pl.ds(base, ew)], idxv)

            def start(b, i):
                pltpu.async_copy(tab_hbm.at[idxv.at[pl.ds(i * gch, gch)]],
                                 bufs[b], sems[b])

            def finish(b, i):
                pltpu.make_async_copy(
                    tab_hbm.at[idxv.at[pl.ds(i * gch, gch)]],
                    bufs[b], sems[b]).wait()
                pltpu.sync_copy(bufs[b],
                                out_hbm.at[pl.ds(base + i * gch, gch)])

            start(0, 0)
            start(1, 1)

            def body(g, carry):
                for b in range(2):
                    i = 2 * g + b
                    finish(b, i)

                    @pl.when(i + 2 < git)
                    def _():
                        start(b, i + 2)
                return carry

            lax.fori_loop(0, git // 2, body, 0)
            if git % 2:
                finish(0, git - 1)

        @pl.when(c == 0)
        def _():
            run(d_hbm, dst_hbm, rd_hbm)

        @pl.when(c == 1)
        def _():
            run(s_hbm, src_hbm, rs_hbm)

    def call(d, s, dst, src):
        k = functools.partial(
            pl.kernel,
            mesh=plsc.VectorSubcoreMesh(core_axis_name="c",
                                        subcore_axis_name="s"),
            out_type=[
                jax.ShapeDtypeStruct((n_edges, F_DIM), jnp.int32),
                jax.ShapeDtypeStruct((n_edges, F_DIM), jnp.int32),
            ],
            scratch_types=[
                pltpu.VMEM((ew,), jnp.int32),
                pltpu.VMEM((gch, F_DIM), jnp.int32),
                pltpu.VMEM((gch, F_DIM), jnp.int32),
                pltpu.SemaphoreType.DMA,
                pltpu.SemaphoreType.DMA,
            ],
        )(_gather_kernel)
        return k(d, s, dst, src)

    return call


_EHALF = N_EDGES // 2
_edge_gather_half = _make_edge_gather(_EHALF, 40)


_EDGE_BLK = 2000


def _msg_body(rd_ref, rs_ref, ea_ref, weg_ref, wec_ref, bg_ref, bc_ref,
              m0_ref, m1_ref):
    wd = rd_ref[...]
    ws = rs_ref[...]
    ea = ea_ref[...]
    g = (_unpack_lo(wd) + _unpack_lo(ws)
         + jnp.dot(ea, weg_ref[...], preferred_element_type=jnp.float32)
         + bg_ref[...])
    c = (_unpack_hi(wd) + _unpack_hi(ws)
         + jnp.dot(ea, wec_ref[...], preferred_element_type=jnp.float32)
         + bc_ref[...])
    gate = 1.0 / (1.0 + jnp.exp(-g))
    sp = jnp.maximum(c, 0.0) + jnp.log(1.0 + jnp.exp(-jnp.abs(c)))
    msg = gate * sp
    m0_ref[...] = msg[:, : F_DIM // 2]
    m1_ref[...] = msg[:, F_DIM // 2:]


def _edge_messages(rd, rs, ea, weg, wec, bg, bc):
    n_edges = rd.shape[0]
    grid = (n_edges // _EDGE_BLK,)
    wspec = pl.BlockSpec((DE_DIM, F_DIM), lambda i: (0, 0))
    bspec = pl.BlockSpec((1, F_DIM), lambda i: (0, 0))
    return pl.pallas_call(
        _msg_body,
        grid=grid,
        in_specs=[
            pl.BlockSpec((_EDGE_BLK, F_DIM), lambda i: (i, 0)),
            pl.BlockSpec((_EDGE_BLK, F_DIM), lambda i: (i, 0)),
            pl.BlockSpec((_EDGE_BLK, DE_DIM), lambda i: (i, 0)),
            wspec, wspec, bspec, bspec,
        ],
        out_specs=[
            pl.BlockSpec((_EDGE_BLK, F_DIM // 2), lambda i: (i, 0)),
            pl.BlockSpec((_EDGE_BLK, F_DIM // 2), lambda i: (i, 0)),
        ],
        out_shape=[
            jax.ShapeDtypeStruct((n_edges, F_DIM // 2), jnp.float32),
            jax.ShapeDtypeStruct((n_edges, F_DIM // 2), jnp.float32),
        ],
    )(rd, rs, ea, weg, wec, bg, bc)


_SCH = 80
_ET = N_EDGES // NS
_SIT = _ET // _SCH
_ZROWS = 1000
_ZTILES = N_NODES // _ZROWS
_HF = F_DIM // 2


_HTILES = NS // 2


def _scatter_kernel(m0a_hbm, m0b_hbm, m1a_hbm, m1b_hbm, dst3d_hbm, z_hbm,
                    a0_hbm, a1_hbm,
                    idxv, mbuf0, mbuf1, acc_sh, seml0, seml1, semc0, semc1):
    c = lax.axis_index("c")
    s = lax.axis_index("s")

    @pl.when(s < _ZTILES)
    def _():
        pltpu.sync_copy(z_hbm, acc_sh.at[pl.ds(s * _ZROWS, _ZROWS)])

    plsc.subcore_barrier()
    pltpu.sync_copy(dst3d_hbm.at[s], idxv)

    mbufs = (mbuf0, mbuf1)
    semls = (seml0, seml1)
    semcs = (semc0, semc1)

    def run(m_hbm, rowbase):
        def startload(b, i):
            pltpu.async_copy(m_hbm.at[pl.ds(rowbase + i * _SCH, _SCH)],
                             mbufs[b], semls[b])

        def finish(b, i):
            pltpu.make_async_copy(
                m_hbm.at[pl.ds(rowbase + i * _SCH, _SCH)],
                mbufs[b], semls[b]).wait()
            pltpu.async_copy(mbufs[b], acc_sh.at[idxv.at[i]], semcs[b],
                             add=True)
            pltpu.make_async_copy(mbufs[b], acc_sh.at[idxv.at[i]],
                                  semcs[b]).wait()

        startload(0, 0)
        startload(1, 1)

        def body(g, carry):
            for b in range(2):
                i = 2 * g + b
                finish(b, i)

                @pl.when(i + 2 < _SIT)
                def _():
                    startload(b, i + 2)
            return carry

        lax.fori_loop(0, _SIT // 2, body, 0)
        if _SIT % 2:
            finish(0, _SIT - 1)

    for cc, grp, m_hbm in ((0, 0, m0a_hbm), (0, 1, m0b_hbm),
                           (1, 0, m1a_hbm), (1, 1, m1b_hbm)):
        lo = grp * _HTILES

        @pl.when(jnp.logical_and(c == cc,
                                 jnp.logical_and(s >= lo, s < lo + _HTILES)))
        def _(m_hbm=m_hbm, lo=lo):
            run(m_hbm, (s - lo) * _ET)

    plsc.subcore_barrier()

    @pl.when(jnp.logical_and(c == 0, s < _ZTILES))
    def _():
        pltpu.sync_copy(acc_sh.at[pl.ds(s * _ZROWS, _ZROWS)],
                        a0_hbm.at[pl.ds(s * _ZROWS, _ZROWS)])

    @pl.when(jnp.logical_and(c == 1, s < _ZTILES))
    def _():
        pltpu.sync_copy(acc_sh.at[pl.ds(s * _ZROWS, _ZROWS)],
                        a1_hbm.at[pl.ds(s * _ZROWS, _ZROWS)])


def _scatter_add(m0a, m0b, m1a, m1b, dst3d, zeros_blk):
    k = functools.partial(
        pl.kernel,
        mesh=plsc.VectorSubcoreMesh(core_axis_name="c", subcore_axis_name="s"),
        out_type=[
            jax.ShapeDtypeStruct((N_NODES, _HF), jnp.float32),
            jax.ShapeDtypeStruct((N_NODES, _HF), jnp.float32),
        ],
        scratch_types=[
            pltpu.VMEM((_SIT, _SCH), jnp.int32),
            pltpu.VMEM((_SCH, _HF), jnp.float32),
            pltpu.VMEM((_SCH, _HF), jnp.float32),
            pltpu.VMEM_SHARED((N_NODES, _HF), jnp.float32),
            pltpu.SemaphoreType.DMA,
            pltpu.SemaphoreType.DMA,
            pltpu.SemaphoreType.DMA,
            pltpu.SemaphoreType.DMA,
        ],
    )(_scatter_kernel)
    return k(m0a, m0b, m1a, m1b, dst3d, zeros_blk)


def _mlp_body(x_ref, a0_ref, a1_ref, wffw_ref, bffw_ref, wproj_ref,
              bproj_ref, o_ref):
    h = x_ref[...] + jnp.concatenate([a0_ref[...], a1_ref[...]], axis=1)
    h = jnp.maximum(
        jnp.dot(h, wffw_ref[...], preferred_element_type=jnp.float32)
        + bffw_ref[...], 0.0)
    o_ref[...] = (jnp.dot(h, wproj_ref[...], preferred_element_type=jnp.float32)
                  + bproj_ref[...])


def _out_mlp(x, a0, a1, wffw, bffw, wproj, bproj):
    grid = (N_NODES // _ROWS_BLK,)
    return pl.pallas_call(
        _mlp_body,
        grid=grid,
        in_specs=[
            pl.BlockSpec((_ROWS_BLK, F_DIM), lambda i: (i, 0)),
            pl.BlockSpec((_ROWS_BLK, _HF), lambda i: (i, 0)),
            pl.BlockSpec((_ROWS_BLK, _HF), lambda i: (i, 0)),
            pl.BlockSpec((F_DIM, F_DIM), lambda i: (0, 0)),
            pl.BlockSpec((1, F_DIM), lambda i: (0, 0)),
            pl.BlockSpec((F_DIM, F_DIM), lambda i: (0, 0)),
            pl.BlockSpec((1, F_DIM), lambda i: (0, 0)),
        ],
        out_specs=pl.BlockSpec((_ROWS_BLK, F_DIM), lambda i: (i, 0)),
        out_shape=jax.ShapeDtypeStruct((N_NODES, F_DIM), jnp.float32),
    )(x, a0, a1, wffw, bffw, wproj, bproj)


def kernel(x, edge_index, edge_attr, Wf, bf, Ws, bs, Wffw, bffw, Wproj, bproj):
    src = edge_index[0]
    dst = edge_index[1]
    d_tab, s_tab = _node_tables(x, Wf[:F_DIM], Ws[:F_DIM],
                                Wf[F_DIM:2 * F_DIM], Ws[F_DIM:2 * F_DIM])
    weg, wec = Wf[2 * F_DIM:], Ws[2 * F_DIM:]
    bg, bc = bf.reshape(1, F_DIM), bs.reshape(1, F_DIM)
    rd0, rs0 = _edge_gather_half(d_tab, s_tab, dst[:_EHALF], src[:_EHALF])
    rd1, rs1 = _edge_gather_half(d_tab, s_tab, dst[_EHALF:], src[_EHALF:])
    m0a, m1a = _edge_messages(rd0, rs0, edge_attr[:_EHALF], weg, wec, bg, bc)
    m0b, m1b = _edge_messages(rd1, rs1, edge_attr[_EHALF:], weg, wec, bg, bc)
    dst3d = dst.reshape(NS, _SIT, _SCH)
    zeros_blk = jnp.zeros((_ZROWS, _HF), jnp.float32)
    a0, a1 = _scatter_add(m0a, m0b, m1a, m1b, dst3d, zeros_blk)
    return _out_mlp(x, a0, a1, bffw=bffw.reshape(1, F_DIM), wffw=Wffw,
                    wproj=Wproj, bproj=bproj.reshape(1, F_DIM))


# speedup vs baseline: 3.1917x; 1.0095x over previous
import functools

import jax
import jax.numpy as jnp
from jax import lax
from jax.experimental import pallas as pl
from jax.experimental.pallas import tpu as pltpu
from jax.experimental.pallas import tpu_sc as plsc

N_NODES = 10000
N_EDGES = 160000
F_DIM = 256
DE_DIM = 16

NC = 2
NS = 16
NW = NC * NS


_ROWS_BLK = 1000


def _rne16(x):
    b = jax.lax.bitcast_convert_type(x, jnp.int32)
    r = b + 0x7FFF + jax.lax.shift_right_logical(b, 16) % 2
    return jax.lax.shift_right_logical(r, 16)


def _pack2(gate_f32, core_f32):
    return _rne16(gate_f32) | (_rne16(core_f32) << 16)


def _unpack_lo(w):
    return jax.lax.bitcast_convert_type(w << 16, jnp.float32)


def _unpack_hi(w):
    return jax.lax.bitcast_convert_type(w & jnp.int32(-65536), jnp.float32)


def _tables_body(x_ref, wfd_ref, wsd_ref, wfs_ref, wss_ref, d_ref, s_ref):
    xb = x_ref[...]
    d_ref[...] = _pack2(
        jnp.dot(xb, wfd_ref[...], preferred_element_type=jnp.float32),
        jnp.dot(xb, wsd_ref[...], preferred_element_type=jnp.float32))
    s_ref[...] = _pack2(
        jnp.dot(xb, wfs_ref[...], preferred_element_type=jnp.float32),
        jnp.dot(xb, wss_ref[...], preferred_element_type=jnp.float32))


def _node_tables(x, wfd, wsd, wfs, wss):
    grid = (N_NODES // _ROWS_BLK,)
    wspec = pl.BlockSpec((F_DIM, F_DIM), lambda i: (0, 0))
    return pl.pallas_call(
        _tables_body,
        grid=grid,
        in_specs=[
            pl.BlockSpec((_ROWS_BLK, F_DIM), lambda i: (i, 0)),
            wspec, wspec, wspec, wspec,
        ],
        out_specs=[
            pl.BlockSpec((_ROWS_BLK, F_DIM), lambda i: (i, 0)),
            pl.BlockSpec((_ROWS_BLK, F_DIM), lambda i: (i, 0)),
        ],
        out_shape=[
            jax.ShapeDtypeStruct((N_NODES, F_DIM), jnp.int32),
            jax.ShapeDtypeStruct((N_NODES, F_DIM), jnp.int32),
        ],
    )(x, wfd, wsd, wfs, wss)


def _make_edge_gather(n_edges, gch):
    ew = n_edges // NS
    git = ew // gch

    def _gather_kernel(d_hbm, s_hbm, dst_hbm, src_hbm, rd_hbm, rs_hbm,
                       idxv, buf0, buf1, sem0, sem1):
        c = lax.axis_index("c")
        s = lax.axis_index("s")
        base = s * ew
        bufs = (buf0, buf1)
        sems = (sem0, sem1)

        def run(tab_hbm, idx_hbm, out_hbm):
            pltpu.sync_copy(idx_hbm.at[pl.ds(base, ew)], idxv)

            def start(b, i):
                pltpu.async_copy(tab_hbm.at[idxv.at[pl.ds(i * gch, gch)]],
                                 bufs[b], sems[b])

            def finish(b, i):
                pltpu.make_async_copy(
                    tab_hbm.at[idxv.at[pl.ds(i * gch, gch)]],
                    bufs[b], sems[b]).wait()
                pltpu.sync_copy(bufs[b],
                                out_hbm.at[pl.ds(base + i * gch, gch)])

            start(0, 0)
            start(1, 1)

            def body(g, carry):
                for b in range(2):
                    i = 2 * g + b
                    finish(b, i)

                    @pl.when(i + 2 < git)
                    def _():
                        start(b, i + 2)
                return carry

            lax.fori_loop(0, git // 2, body, 0)
            if git % 2:
                finish(0, git - 1)

        @pl.when(c == 0)
        def _():
            run(d_hbm, dst_hbm, rd_hbm)

        @pl.when(c == 1)
        def _():
            run(s_hbm, src_hbm, rs_hbm)

    def call(d, s, dst, src):
        k = functools.partial(
            pl.kernel,
            mesh=plsc.VectorSubcoreMesh(core_axis_name="c",
                                        subcore_axis_name="s"),
            out_type=[
                jax.ShapeDtypeStruct((n_edges, F_DIM), jnp.int32),
                jax.ShapeDtypeStruct((n_edges, F_DIM), jnp.int32),
            ],
            scratch_types=[
                pltpu.VMEM((ew,), jnp.int32),
                pltpu.VMEM((gch, F_DIM), jnp.int32),
                pltpu.VMEM((gch, F_DIM), jnp.int32),
                pltpu.SemaphoreType.DMA,
                pltpu.SemaphoreType.DMA,
            ],
        )(_gather_kernel)
        return k(d, s, dst, src)

    return call


_EHALF = N_EDGES // 2
_edge_gather_half = _make_edge_gather(_EHALF, 40)


_EDGE_BLK = 2000


def _msg_body(rd_ref, rs_ref, ea_ref, weg_ref, wec_ref, bg_ref, bc_ref,
              m0_ref, m1_ref):
    wd = rd_ref[...]
    ws = rs_ref[...]
    ea = ea_ref[...]
    g = (_unpack_lo(wd) + _unpack_lo(ws)
         + jnp.dot(ea, weg_ref[...], preferred_element_type=jnp.float32)
         + bg_ref[...])
    c = (_unpack_hi(wd) + _unpack_hi(ws)
         + jnp.dot(ea, wec_ref[...], preferred_element_type=jnp.float32)
         + bc_ref[...])
    gate = 1.0 / (1.0 + jnp.exp(-g))
    sp = jnp.maximum(c, 0.0) + jnp.log(1.0 + jnp.exp(-jnp.abs(c)))
    msg = gate * sp
    m0_ref[...] = msg[:, : F_DIM // 2]
    m1_ref[...] = msg[:, F_DIM // 2:]


def _edge_messages(rd, rs, ea, weg, wec, bg, bc):
    n_edges = rd.shape[0]
    grid = (n_edges // _EDGE_BLK,)
    wspec = pl.BlockSpec((DE_DIM, F_DIM), lambda i: (0, 0))
    bspec = pl.BlockSpec((1, F_DIM), lambda i: (0, 0))
    return pl.pallas_call(
        _msg_body,
        grid=grid,
        in_specs=[
            pl.BlockSpec((_EDGE_BLK, F_DIM), lambda i: (i, 0)),
            pl.BlockSpec((_EDGE_BLK, F_DIM), lambda i: (i, 0)),
            pl.BlockSpec((_EDGE_BLK, DE_DIM), lambda i: (i, 0)),
            wspec, wspec, bspec, bspec,
        ],
        out_specs=[
            pl.BlockSpec((_EDGE_BLK, F_DIM // 2), lambda i: (i, 0)),
            pl.BlockSpec((_EDGE_BLK, F_DIM // 2), lambda i: (i, 0)),
        ],
        out_shape=[
            jax.ShapeDtypeStruct((n_edges, F_DIM // 2), jnp.float32),
            jax.ShapeDtypeStruct((n_edges, F_DIM // 2), jnp.float32),
        ],
    )(rd, rs, ea, weg, wec, bg, bc)


_ZROWS = 1000
_ZTILES = N_NODES // _ZROWS
_HF = F_DIM // 2


def _make_scatter_add(n_edges, sch):
    et = n_edges // NS
    sit = et // sch

    def _scatter_kernel(m0_hbm, m1_hbm, dst3d_hbm, z0_hbm, z1_hbm,
                        a0_hbm, a1_hbm,
                        idxv, mbuf0, mbuf1, acc_sh,
                        seml0, seml1, semc0, semc1):
        c = lax.axis_index("c")
        s = lax.axis_index("s")

        @pl.when(jnp.logical_and(c == 0, s < _ZTILES))
        def _():
            pltpu.sync_copy(z0_hbm.at[pl.ds(s * _ZROWS, _ZROWS)],
                            acc_sh.at[pl.ds(s * _ZROWS, _ZROWS)])

        @pl.when(jnp.logical_and(c == 1, s < _ZTILES))
        def _():
            pltpu.sync_copy(z1_hbm.at[pl.ds(s * _ZROWS, _ZROWS)],
                            acc_sh.at[pl.ds(s * _ZROWS, _ZROWS)])

        plsc.subcore_barrier()
        pltpu.sync_copy(dst3d_hbm.at[s], idxv)

        mbufs = (mbuf0, mbuf1)
        semls = (seml0, seml1)
        semcs = (semc0, semc1)

        def run(m_hbm):
            def startload(b, i):
                pltpu.async_copy(m_hbm.at[pl.ds(s * et + i * sch, sch)],
                                 mbufs[b], semls[b])

            def finish(b, i):
                pltpu.make_async_copy(
                    m_hbm.at[pl.ds(s * et + i * sch, sch)],
                    mbufs[b], semls[b]).wait()
                pltpu.async_copy(mbufs[b], acc_sh.at[idxv.at[i]], semcs[b],
                                 add=True)
                pltpu.make_async_copy(mbufs[b], acc_sh.at[idxv.at[i]],
                                      semcs[b]).wait()

            startload(0, 0)
            startload(1, 1)

            def body(g, carry):
                for b in range(2):
                    i = 2 * g + b
                    finish(b, i)

                    @pl.when(i + 2 < sit)
                    def _():
                        startload(b, i + 2)
                return carry

            lax.fori_loop(0, sit // 2, body, 0)
            if sit % 2:
                finish(0, sit - 1)

        @pl.when(c == 0)
        def _():
            run(m0_hbm)

        @pl.when(c == 1)
        def _():
            run(m1_hbm)

        plsc.subcore_barrier()

        @pl.when(jnp.logical_and(c == 0, s < _ZTILES))
        def _():
            pltpu.sync_copy(acc_sh.at[pl.ds(s * _ZROWS, _ZROWS)],
                            a0_hbm.at[pl.ds(s * _ZROWS, _ZROWS)])

        @pl.when(jnp.logical_and(c == 1, s < _ZTILES))
        def _():
            pltpu.sync_copy(acc_sh.at[pl.ds(s * _ZROWS, _ZROWS)],
                            a1_hbm.at[pl.ds(s * _ZROWS, _ZROWS)])

    def call(m0, m1, dst3d, z0, z1):
        k = functools.partial(
            pl.kernel,
            mesh=plsc.VectorSubcoreMesh(core_axis_name="c",
                                        subcore_axis_name="s"),
            out_type=[
                jax.ShapeDtypeStruct((N_NODES, _HF), jnp.float32),
                jax.ShapeDtypeStruct((N_NODES, _HF), jnp.float32),
            ],
            scratch_types=[
                pltpu.VMEM((sit, sch), jnp.int32),
                pltpu.VMEM((sch, _HF), jnp.float32),
                pltpu.VMEM((sch, _HF), jnp.float32),
                pltpu.VMEM_SHARED((N_NODES, _HF), jnp.float32),
                pltpu.SemaphoreType.DMA,
                pltpu.SemaphoreType.DMA,
                pltpu.SemaphoreType.DMA,
                pltpu.SemaphoreType.DMA,
            ],
        )(_scatter_kernel)
        return k(m0, m1, dst3d, z0, z1)

    return call


_SCH = 40
_scatter_add_half = _make_scatter_add(_EHALF, _SCH)


def _mlp_body(x_ref, a0_ref, a1_ref, wffw_ref, bffw_ref, wproj_ref,
              bproj_ref, o_ref):
    h = x_ref[...] + jnp.concatenate([a0_ref[...], a1_ref[...]], axis=1)
    h = jnp.maximum(
        jnp.dot(h, wffw_ref[...], preferred_element_type=jnp.float32)
        + bffw_ref[...], 0.0)
    o_ref[...] = (jnp.dot(h, wproj_ref[...], preferred_element_type=jnp.float32)
                  + bproj_ref[...])


def _out_mlp(x, a0, a1, wffw, bffw, wproj, bproj):
    grid = (N_NODES // _ROWS_BLK,)
    return pl.pallas_call(
        _mlp_body,
        grid=grid,
        in_specs=[
            pl.BlockSpec((_ROWS_BLK, F_DIM), lambda i: (i, 0)),
            pl.BlockSpec((_ROWS_BLK, _HF), lambda i: (i, 0)),
            pl.BlockSpec((_ROWS_BLK, _HF), lambda i: (i, 0)),
            pl.BlockSpec((F_DIM, F_DIM), lambda i: (0, 0)),
            pl.BlockSpec((1, F_DIM), lambda i: (0, 0)),
            pl.BlockSpec((F_DIM, F_DIM), lambda i: (0, 0)),
            pl.BlockSpec((1, F_DIM), lambda i: (0, 0)),
        ],
        out_specs=pl.BlockSpec((_ROWS_BLK, F_DIM), lambda i: (i, 0)),
        out_shape=jax.ShapeDtypeStruct((N_NODES, F_DIM), jnp.float32),
    )(x, a0, a1, wffw, bffw, wproj, bproj)


def kernel(x, edge_index, edge_attr, Wf, bf, Ws, bs, Wffw, bffw, Wproj, bproj):
    src = edge_index[0]
    dst = edge_index[1]
    d_tab, s_tab = _node_tables(x, Wf[:F_DIM], Ws[:F_DIM],
                                Wf[F_DIM:2 * F_DIM], Ws[F_DIM:2 * F_DIM])
    weg, wec = Wf[2 * F_DIM:], Ws[2 * F_DIM:]
    bg, bc = bf.reshape(1, F_DIM), bs.reshape(1, F_DIM)
    rd0, rs0 = _edge_gather_half(d_tab, s_tab, dst[:_EHALF], src[:_EHALF])
    rd1, rs1 = _edge_gather_half(d_tab, s_tab, dst[_EHALF:], src[_EHALF:])
    m0a, m1a = _edge_messages(rd0, rs0, edge_attr[:_EHALF], weg, wec, bg, bc)
    m0b, m1b = _edge_messages(rd1, rs1, edge_attr[_EHALF:], weg, wec, bg, bc)
    sit = (_EHALF // NS) // _SCH
    dst3d_a = dst[:_EHALF].reshape(NS, sit, _SCH)
    dst3d_b = dst[_EHALF:].reshape(NS, sit, _SCH)
    zeros_full = jnp.zeros((N_NODES, _HF), jnp.float32)
    a0p, a1p = _scatter_add_half(m0a, m1a, dst3d_a, zeros_full, zeros_full)
    a0, a1 = _scatter_add_half(m0b, m1b, dst3d_b, a0p, a1p)
    return _out_mlp(x, a0, a1, bffw=bffw.reshape(1, F_DIM), wffw=Wffw,
                    wproj=Wproj, bproj=bproj.reshape(1, F_DIM))
